# baseline (speedup 1.0000x reference)
import functools

import jax
import jax.numpy as jnp
from jax import lax
from jax.experimental import pallas as pl
from jax.experimental.pallas import tpu as pltpu
from jax.experimental.pallas import tpu_sc as plsc

N = 10000
E = 320000
F_IN = 128
HID = 128
OUT = 64
ALPHA = 0.1
KSTEPS = 2

NC = 2
NS = 16
LANES = 16

NP = 10240
ROWS_PT = NP // NS
CHUNK = 128
CPT = 160
EPAD = NS * CPT * CHUNK
SUB = 80
NSUB = ROWS_PT // SUB


@functools.lru_cache(maxsize=None)
def _get_mesh():
    return plsc.VectorSubcoreMesh(
        core_axis_name="c", subcore_axis_name="s",
        num_cores=NC, num_subcores=NS,
    )


def _rsqrt16(d):
    i = lax.bitcast_convert_type(d, jnp.int32)
    i = jnp.int32(0x5F3759DF) - (i >> 1)
    y = lax.bitcast_convert_type(i, jnp.float32)
    for _ in range(3):
        y = y * (1.5 - 0.5 * d * y * y)
    return y


def _degree_body(eidx_hbm, rv_hbm, acc_sp, idx_v, ones_v, row_v):
    c = lax.axis_index("c")
    s = lax.axis_index("s")
    base = s * ROWS_PT

    for k in range(CHUNK // LANES):
        ones_v[pl.ds(k * LANES, LANES)] = jnp.ones((LANES,), jnp.float32)

    def zloop(k, carry):
        row_v[pl.ds(k * LANES, LANES)] = jnp.zeros((LANES,), jnp.float32)
        return carry

    lax.fori_loop(0, ROWS_PT // LANES, zloop, None)
    pltpu.sync_copy(row_v, acc_sp.at[pl.ds(base, ROWS_PT)])
    pltpu.sync_copy(eidx_hbm.at[pl.ds((c * NS + s) * CPT, CPT)], idx_v)
    plsc.subcore_barrier()

    def eloop(j, carry):
        pltpu.sync_copy(ones_v, acc_sp.at[idx_v.at[j]], add=True)
        return carry

    lax.fori_loop(0, CPT, eloop, None)
    plsc.subcore_barrier()

    pltpu.sync_copy(acc_sp.at[pl.ds(base, ROWS_PT)], row_v)

    def rloop(k, carry):
        sl = pl.ds(k * LANES, LANES)
        row_v[sl] = _rsqrt16(jnp.maximum(row_v[sl], 1.0))
        return carry

    lax.fori_loop(0, ROWS_PT // LANES, rloop, None)
    pltpu.sync_copy(row_v, rv_hbm.at[c, pl.ds(base, ROWS_PT)])


@functools.lru_cache(maxsize=None)
def _get_degree_kernel():
    return pl.kernel(
        _degree_body,
        out_type=jax.ShapeDtypeStruct((NC, NP), jnp.float32),
        mesh=_get_mesh(),
        scratch_types=[
            pltpu.VMEM_SHARED((NP,), jnp.float32),
            pltpu.VMEM((CPT, CHUNK), jnp.int32),
            pltpu.VMEM((CHUNK,), jnp.float32),
            pltpu.VMEM((ROWS_PT,), jnp.float32),
        ],
    )


@functools.lru_cache(maxsize=None)
def _make_prop(F):
    Fh = F // NC
    nv = Fh // LANES
    NR = 2

    @functools.partial(
        pl.kernel,
        out_type=jax.ShapeDtypeStruct((NC, NP, Fh), jnp.float32),
        mesh=_get_mesh(),
        compiler_params=pltpu.CompilerParams(use_tc_tiling_on_sc=False),
        scratch_types=[
            pltpu.VMEM_SHARED((NP, Fh), jnp.float32),
            pltpu.VMEM_SHARED((NP, Fh), jnp.float32),
            pltpu.VMEM((2, 2, CHUNK), jnp.int32),
            pltpu.VMEM((2, 2, CHUNK), jnp.int32),
            pltpu.VMEM((CHUNK, Fh), jnp.float32),
            pltpu.VMEM((CHUNK, Fh), jnp.float32),
            pltpu.VMEM((SUB, Fh), jnp.float32),
            pltpu.VMEM((SUB, Fh), jnp.float32),
            pltpu.VMEM((SUB,), jnp.float32),
            pltpu.VMEM((SUB,), jnp.float32),
            pltpu.SemaphoreType.DMA,
            pltpu.SemaphoreType.DMA,
            pltpu.SemaphoreType.DMA,
            pltpu.SemaphoreType.DMA,
            pltpu.SemaphoreType.DMA,
            pltpu.SemaphoreType.DMA,
        ],
    )
    def _prop(h_hbm, rv_hbm, eidx_hbm, z_hbm,
              acc_sp, u_sp, ix0, ix1, rb0, rb1,
              abuf, hbuf, av, bv, sem0, sem1,
              is00, is01, is10, is11):
        c = lax.axis_index("c")
        s = lax.axis_index("s")
        base = s * ROWS_PT
        ebase = s * CPT

        ixs = (ix0, ix1)
        isems = ((is00, is01), (is10, is11))

        def init_sub(i, carry):
            r0 = base + i * SUB
            pltpu.sync_copy(h_hbm.at[c, pl.ds(r0, SUB)], hbuf)
            pltpu.sync_copy(rv_hbm.at[0, pl.ds(r0, SUB)], av)

            def rowl(g, carry2):
                r0g = g * LANES
                a16 = av[pl.ds(r0g, LANES)]
                for i2 in range(LANES):
                    a = a16[i2]
                    for v in range(nv):
                        sl = pl.ds(v * LANES, LANES)
                        hbuf[r0g + i2, sl] = a * hbuf[r0g + i2, sl]
                return carry2

            lax.fori_loop(0, SUB // LANES, rowl, None)
            pltpu.sync_copy(hbuf, u_sp.at[pl.ds(r0, SUB)])
            return carry

        lax.fori_loop(0, NSUB, init_sub, None)

        rbufs = (rb0, rb1)
        sems = (sem0, sem1)

        for step in range(KSTEPS):
            last = step == KSTEPS - 1

            ZB = 128

            def zl(k, carry):
                for v in range(nv):
                    rb0[k, pl.ds(v * LANES, LANES)] = jnp.zeros(
                        (LANES,), jnp.float32)
                return carry

            lax.fori_loop(0, ZB, zl, None)

            def zc(i, carry):
                pltpu.sync_copy(
                    rb0.at[pl.ds(0, ZB)],
                    acc_sp.at[pl.ds(base + i * ZB, ZB)])
                return carry

            lax.fori_loop(0, ROWS_PT // ZB, zc, None)
            plsc.subcore_barrier()

            for r in range(NR):
                pltpu.async_copy(
                    eidx_hbm.at[ebase + r], ixs[r].at[0], isems[r][0])
            for r in range(NR):
                pltpu.async_copy(
                    eidx_hbm.at[ebase + NR + r], ixs[r].at[1], isems[r][1])
            for r in range(NR):
                pltpu.make_async_copy(
                    eidx_hbm.at[ebase], ixs[r].at[0], isems[r][0]).wait()
                pltpu.async_copy(
                    u_sp.at[ixs[r].at[0, 0]], rbufs[r], sems[r])

            def el(g, carry):
                for half in range(2):
                    bx, by = (0, 1) if half == 0 else (1, 0)
                    for r in range(NR):
                        j = g * 2 * NR + half * NR + r
                        pltpu.make_async_copy(
                            u_sp.at[ixs[r].at[0, 0]], rbufs[r],
                            sems[r]).wait()
                        pltpu.sync_copy(
                            rbufs[r], acc_sp.at[ixs[r].at[bx, 1]], add=True)

                        @pl.when(j + 2 * NR < CPT)
                        def _prefetch_idx():
                            pltpu.async_copy(
                                eidx_hbm.at[ebase + j + 2 * NR],
                                ixs[r].at[bx], isems[r][bx])

                        @pl.when(j + NR < CPT)
                        def _refill():
                            pltpu.make_async_copy(
                                eidx_hbm.at[ebase], ixs[r].at[by],
                                isems[r][by]).wait()
                            pltpu.async_copy(
                                u_sp.at[ixs[r].at[by, 0]], rbufs[r], sems[r])

                return carry

            lax.fori_loop(0, CPT // (2 * NR), el, None)
            plsc.subcore_barrier()

            def comb_sub(i, carry):
                r0 = base + i * SUB
                pltpu.sync_copy(acc_sp.at[pl.ds(r0, SUB)], abuf)
                pltpu.sync_copy(h_hbm.at[c, pl.ds(r0, SUB)], hbuf)
                pltpu.sync_copy(rv_hbm.at[0, pl.ds(r0, SUB)], av)
                pltpu.sync_copy(rv_hbm.at[1, pl.ds(r0, SUB)], bv)

                def rowl(g, carry2):
                    r0g = g * LANES
                    a16 = av[pl.ds(r0g, LANES)]
                    b16 = bv[pl.ds(r0g, LANES)] * (1.0 - ALPHA)
                    for i2 in range(LANES):
                        a = a16[i2]
                        bb = b16[i2]
                        r = r0g + i2
                        for v in range(nv):
                            sl = pl.ds(v * LANES, LANES)
                            z = bb * abuf[r, sl] + ALPHA * hbuf[r, sl]
                            hbuf[r, sl] = z
                            abuf[r, sl] = a * z
                    return carry2

                lax.fori_loop(0, SUB // LANES, rowl, None)
                if last:
                    pltpu.sync_copy(hbuf, z_hbm.at[c, pl.ds(r0, SUB)])
                else:
                    pltpu.sync_copy(abuf, u_sp.at[pl.ds(r0, SUB)])
                return carry

            lax.fori_loop(0, NSUB, comb_sub, None)
            plsc.subcore_barrier()

    return _prop


def _mm1(x, W, bvec):
    Fh = HID // NC
    BM = 1280

    def body(x_ref, w_ref, b_ref, o_ref):
        c = pl.program_id(1)
        w = w_ref[...]
        b = b_ref[...]
        wk = jnp.where(c == 0, w[:, :Fh], w[:, Fh:])
        bk = jnp.where(c == 0, b[:, :Fh], b[:, Fh:])
        o_ref[0] = (
            jnp.dot(x_ref[...], wk, preferred_element_type=jnp.float32) + bk
        )

    return pl.pallas_call(
        body,
        grid=(NP // BM, NC),
        in_specs=[
            pl.BlockSpec((BM, F_IN), lambda i, c: (i, 0)),
            pl.BlockSpec((F_IN, HID), lambda i, c: (0, 0)),
            pl.BlockSpec((1, HID), lambda i, c: (0, 0)),
        ],
        out_specs=pl.BlockSpec((1, BM, Fh), lambda i, c: (c, i, 0)),
        out_shape=jax.ShapeDtypeStruct((NC, NP, Fh), jnp.float32),
    )(x, W, bvec.reshape(1, HID))


def _mm2(zs, W, bvec):
    Kh = HID // NC
    Fh = OUT // NC
    BM = 1280

    def body(z_ref, w_ref, b_ref, o_ref):
        c = pl.program_id(1)
        w = w_ref[...]
        b = b_ref[...]
        wc = jnp.where(c == 0, w[:, :Fh], w[:, Fh:])
        bc = jnp.where(c == 0, b[:, :Fh], b[:, Fh:])
        out = jnp.zeros((BM, Fh), jnp.float32) + bc
        for k in range(NC):
            xk = jnp.maximum(z_ref[k], 0.0)
            wk = wc[k * Kh:(k + 1) * Kh, :]
            out = out + jnp.dot(xk, wk, preferred_element_type=jnp.float32)
        o_ref[0] = out

    return pl.pallas_call(
        body,
        grid=(NP // BM, NC),
        in_specs=[
            pl.BlockSpec((NC, BM, Kh), lambda i, c: (0, i, 0)),
            pl.BlockSpec((HID, OUT), lambda i, c: (0, 0)),
            pl.BlockSpec((1, OUT), lambda i, c: (0, 0)),
        ],
        out_specs=pl.BlockSpec((1, BM, Fh), lambda i, c: (c, i, 0)),
        out_shape=jax.ShapeDtypeStruct((NC, NP, Fh), jnp.float32),
    )(zs, W, bvec.reshape(1, OUT))


def kernel(x, edge_index, W1, b1, W2, b2):
    src = edge_index[0]
    dst = edge_index[1]
    pad = jnp.full((EPAD - E,), N, jnp.int32)
    sidx = jnp.concatenate([src, pad]).reshape(NS * CPT, CHUNK)
    didx = jnp.concatenate([dst, pad]).reshape(NS * CPT, CHUNK)
    eidx = jnp.concatenate([sidx, didx], axis=0)
    eidx2 = jnp.stack([sidx, didx], axis=1)
    xp = jnp.zeros((NP, F_IN), jnp.float32).at[:N].set(x)

    rv = _get_degree_kernel()(eidx)
    h1 = _mm1(xp, W1, b1)
    z1 = _make_prop(HID)(h1, rv, eidx2)
    h2 = _mm2(z1, W2, b2)
    z2 = _make_prop(OUT)(h2, rv, eidx2)
    out = jnp.concatenate([z2[0], z2[1]], axis=1)
    return out[:N]


# speedup vs baseline: 1.0446x; 1.0446x over previous
import functools

import jax
import jax.numpy as jnp
from jax import lax
from jax.experimental import pallas as pl
from jax.experimental.pallas import tpu as pltpu
from jax.experimental.pallas import tpu_sc as plsc

N = 10000
E = 320000
F_IN = 128
HID = 128
OUT = 64
ALPHA = 0.1
KSTEPS = 2

NC = 2
NS = 16
LANES = 16

NP = 10240
ROWS_PT = NP // NS
CHUNK = 128
CPT = 160
EPAD = NS * CPT * CHUNK
SUB = 80
NSUB = ROWS_PT // SUB


@functools.lru_cache(maxsize=None)
def _get_mesh():
    return plsc.VectorSubcoreMesh(
        core_axis_name="c", subcore_axis_name="s",
        num_cores=NC, num_subcores=NS,
    )


def _rsqrt16(d):
    i = lax.bitcast_convert_type(d, jnp.int32)
    i = jnp.int32(0x5F3759DF) - (i >> 1)
    y = lax.bitcast_convert_type(i, jnp.float32)
    for _ in range(3):
        y = y * (1.5 - 0.5 * d * y * y)
    return y


def _degree_body(eidx_hbm, rv_hbm, acc_sp, idx_v, ones_v, row_v):
    c = lax.axis_index("c")
    s = lax.axis_index("s")
    base = s * ROWS_PT

    for k in range(CHUNK // LANES):
        ones_v[pl.ds(k * LANES, LANES)] = jnp.ones((LANES,), jnp.float32)

    def zloop(k, carry):
        row_v[pl.ds(k * LANES, LANES)] = jnp.zeros((LANES,), jnp.float32)
        return carry

    lax.fori_loop(0, ROWS_PT // LANES, zloop, None)
    pltpu.sync_copy(row_v, acc_sp.at[pl.ds(base, ROWS_PT)])
    pltpu.sync_copy(eidx_hbm.at[pl.ds((c * NS + s) * CPT, CPT)], idx_v)
    plsc.subcore_barrier()

    def eloop(j, carry):
        pltpu.sync_copy(ones_v, acc_sp.at[idx_v.at[j]], add=True)
        return carry

    lax.fori_loop(0, CPT, eloop, None)
    plsc.subcore_barrier()

    pltpu.sync_copy(acc_sp.at[pl.ds(base, ROWS_PT)], row_v)

    def rloop(k, carry):
        sl = pl.ds(k * LANES, LANES)
        row_v[sl] = _rsqrt16(jnp.maximum(row_v[sl], 1.0))
        return carry

    lax.fori_loop(0, ROWS_PT // LANES, rloop, None)
    pltpu.sync_copy(row_v, rv_hbm.at[c, pl.ds(base, ROWS_PT)])


@functools.lru_cache(maxsize=None)
def _get_degree_kernel():
    return pl.kernel(
        _degree_body,
        out_type=jax.ShapeDtypeStruct((NC, NP), jnp.float32),
        mesh=_get_mesh(),
        scratch_types=[
            pltpu.VMEM_SHARED((NP,), jnp.float32),
            pltpu.VMEM((CPT, CHUNK), jnp.int32),
            pltpu.VMEM((CHUNK,), jnp.float32),
            pltpu.VMEM((ROWS_PT,), jnp.float32),
        ],
    )


@functools.lru_cache(maxsize=None)
def _make_prop(F, merge_out=False):
    Fh = F // NC
    nv = Fh // LANES
    NR = 4

    @functools.partial(
        pl.kernel,
        out_type=jax.ShapeDtypeStruct(
            (NP, F) if merge_out else (NC, NP, Fh), jnp.float32),
        mesh=_get_mesh(),
        compiler_params=pltpu.CompilerParams(use_tc_tiling_on_sc=False),
        scratch_types=[
            pltpu.VMEM_SHARED((NP, Fh), jnp.float32),
            pltpu.VMEM_SHARED((NP, Fh), jnp.float32),
            pltpu.VMEM((2, 2, CHUNK), jnp.int32),
            pltpu.VMEM((2, 2, CHUNK), jnp.int32),
            pltpu.VMEM((2, 2, CHUNK), jnp.int32),
            pltpu.VMEM((2, 2, CHUNK), jnp.int32),
            pltpu.VMEM((CHUNK, Fh), jnp.float32),
            pltpu.VMEM((CHUNK, Fh), jnp.float32),
            pltpu.VMEM((CHUNK, Fh), jnp.float32),
            pltpu.VMEM((CHUNK, Fh), jnp.float32),
            pltpu.VMEM((SUB, Fh), jnp.float32),
            pltpu.VMEM((SUB, Fh), jnp.float32),
            pltpu.VMEM((SUB,), jnp.float32),
            pltpu.VMEM((SUB,), jnp.float32),
            pltpu.SemaphoreType.DMA,
            pltpu.SemaphoreType.DMA,
            pltpu.SemaphoreType.DMA,
            pltpu.SemaphoreType.DMA,
            pltpu.SemaphoreType.DMA,
            pltpu.SemaphoreType.DMA,
            pltpu.SemaphoreType.DMA,
            pltpu.SemaphoreType.DMA,
            pltpu.SemaphoreType.DMA,
            pltpu.SemaphoreType.DMA,
            pltpu.SemaphoreType.DMA,
            pltpu.SemaphoreType.DMA,
        ],
    )
    def _prop(h_hbm, rv_hbm, eidx_hbm, z_hbm,
              acc_sp, u_sp, ix0, ix1, ix2, ix3, rb0, rb1, rb2, rb3,
              abuf, hbuf, av, bv, sem0, sem1, sem2, sem3,
              is00, is01, is10, is11, is20, is21, is30, is31):
        c = lax.axis_index("c")
        s = lax.axis_index("s")
        base = s * ROWS_PT
        ebase = s * CPT

        ixs = (ix0, ix1, ix2, ix3)
        isems = ((is00, is01), (is10, is11), (is20, is21), (is30, is31))

        def init_sub(i, carry):
            r0 = base + i * SUB
            pltpu.sync_copy(h_hbm.at[c, pl.ds(r0, SUB)], hbuf)
            pltpu.sync_copy(rv_hbm.at[0, pl.ds(r0, SUB)], av)

            def rowl(g, carry2):
                r0g = g * LANES
                a16 = av[pl.ds(r0g, LANES)]
                for i2 in range(LANES):
                    a = a16[i2]
                    for v in range(nv):
                        sl = pl.ds(v * LANES, LANES)
                        hbuf[r0g + i2, sl] = a * hbuf[r0g + i2, sl]
                return carry2

            lax.fori_loop(0, SUB // LANES, rowl, None)
            pltpu.sync_copy(hbuf, u_sp.at[pl.ds(r0, SUB)])
            return carry

        lax.fori_loop(0, NSUB, init_sub, None)

        rbufs = (rb0, rb1, rb2, rb3)
        sems = (sem0, sem1, sem2, sem3)

        for step in range(KSTEPS):
            last = step == KSTEPS - 1

            ZB = 128

            def zl(k, carry):
                for v in range(nv):
                    rb0[k, pl.ds(v * LANES, LANES)] = jnp.zeros(
                        (LANES,), jnp.float32)
                return carry

            lax.fori_loop(0, ZB, zl, None)

            def zc(i, carry):
                pltpu.sync_copy(
                    rb0.at[pl.ds(0, ZB)],
                    acc_sp.at[pl.ds(base + i * ZB, ZB)])
                return carry

            lax.fori_loop(0, ROWS_PT // ZB, zc, None)
            plsc.subcore_barrier()

            for r in range(NR):
                pltpu.async_copy(
                    eidx_hbm.at[ebase + r], ixs[r].at[0], isems[r][0])
            for r in range(NR):
                pltpu.async_copy(
                    eidx_hbm.at[ebase + NR + r], ixs[r].at[1], isems[r][1])
            for r in range(NR):
                pltpu.make_async_copy(
                    eidx_hbm.at[ebase], ixs[r].at[0], isems[r][0]).wait()
                pltpu.async_copy(
                    u_sp.at[ixs[r].at[0, 0]], rbufs[r], sems[r])

            def el(g, carry):
                for half in range(2):
                    bx, by = (0, 1) if half == 0 else (1, 0)
                    for r in range(NR):
                        j = g * 2 * NR + half * NR + r
                        pltpu.make_async_copy(
                            u_sp.at[ixs[r].at[0, 0]], rbufs[r],
                            sems[r]).wait()
                        pltpu.sync_copy(
                            rbufs[r], acc_sp.at[ixs[r].at[bx, 1]], add=True)

                        @pl.when(j + 2 * NR < CPT)
                        def _prefetch_idx():
                            pltpu.async_copy(
                                eidx_hbm.at[ebase + j + 2 * NR],
                                ixs[r].at[bx], isems[r][bx])

                        @pl.when(j + NR < CPT)
                        def _refill():
                            pltpu.make_async_copy(
                                eidx_hbm.at[ebase], ixs[r].at[by],
                                isems[r][by]).wait()
                            pltpu.async_copy(
                                u_sp.at[ixs[r].at[by, 0]], rbufs[r], sems[r])

                return carry

            lax.fori_loop(0, CPT // (2 * NR), el, None)
            plsc.subcore_barrier()

            def comb_sub(i, carry):
                r0 = base + i * SUB
                pltpu.sync_copy(acc_sp.at[pl.ds(r0, SUB)], abuf)
                pltpu.sync_copy(h_hbm.at[c, pl.ds(r0, SUB)], hbuf)
                pltpu.sync_copy(rv_hbm.at[0, pl.ds(r0, SUB)], av)
                pltpu.sync_copy(rv_hbm.at[1, pl.ds(r0, SUB)], bv)

                def rowl(g, carry2):
                    r0g = g * LANES
                    a16 = av[pl.ds(r0g, LANES)]
                    b16 = bv[pl.ds(r0g, LANES)] * (1.0 - ALPHA)
                    for i2 in range(LANES):
                        a = a16[i2]
                        bb = b16[i2]
                        r = r0g + i2
                        for v in range(nv):
                            sl = pl.ds(v * LANES, LANES)
                            z = bb * abuf[r, sl] + ALPHA * hbuf[r, sl]
                            hbuf[r, sl] = z
                            abuf[r, sl] = a * z
                    return carry2

                lax.fori_loop(0, SUB // LANES, rowl, None)
                if last:
                    if merge_out:
                        pltpu.sync_copy(
                            hbuf,
                            z_hbm.at[pl.ds(r0, SUB), pl.ds(c * Fh, Fh)])
                    else:
                        pltpu.sync_copy(hbuf, z_hbm.at[c, pl.ds(r0, SUB)])
                else:
                    pltpu.sync_copy(abuf, u_sp.at[pl.ds(r0, SUB)])
                return carry

            lax.fori_loop(0, NSUB, comb_sub, None)
            plsc.subcore_barrier()

    return _prop


def _mm1(x, W, bvec):
    Fh = HID // NC
    BM = 1280

    def body(x_ref, w_ref, b_ref, o_ref):
        x = x_ref[...]
        w = w_ref[...]
        b = b_ref[...]
        for cc in range(NC):
            sl = slice(cc * Fh, (cc + 1) * Fh)
            o_ref[cc] = (
                jnp.dot(x, w[:, sl], preferred_element_type=jnp.float32)
                + b[:, sl]
            )

    return pl.pallas_call(
        body,
        grid=(NP // BM,),
        in_specs=[
            pl.BlockSpec((BM, F_IN), lambda i: (i, 0)),
            pl.BlockSpec((F_IN, HID), lambda i: (0, 0)),
            pl.BlockSpec((1, HID), lambda i: (0, 0)),
        ],
        out_specs=pl.BlockSpec((NC, BM, Fh), lambda i: (0, i, 0)),
        out_shape=jax.ShapeDtypeStruct((NC, NP, Fh), jnp.float32),
    )(x, W, bvec.reshape(1, HID))


def _mm2(zs, W, bvec):
    Kh = HID // NC
    Fh = OUT // NC
    BM = 1280

    def body(z_ref, w_ref, b_ref, o_ref):
        w = w_ref[...]
        b = b_ref[...]
        zk = [jnp.maximum(z_ref[k], 0.0) for k in range(NC)]
        for cc in range(NC):
            sl = slice(cc * Fh, (cc + 1) * Fh)
            out = jnp.zeros((BM, Fh), jnp.float32) + b[:, sl]
            for k in range(NC):
                out = out + jnp.dot(
                    zk[k], w[k * Kh:(k + 1) * Kh, sl],
                    preferred_element_type=jnp.float32)
            o_ref[cc] = out

    return pl.pallas_call(
        body,
        grid=(NP // BM,),
        in_specs=[
            pl.BlockSpec((NC, BM, Kh), lambda i: (0, i, 0)),
            pl.BlockSpec((HID, OUT), lambda i: (0, 0)),
            pl.BlockSpec((1, OUT), lambda i: (0, 0)),
        ],
        out_specs=pl.BlockSpec((NC, BM, Fh), lambda i: (0, i, 0)),
        out_shape=jax.ShapeDtypeStruct((NC, NP, Fh), jnp.float32),
    )(zs, W, bvec.reshape(1, OUT))


def kernel(x, edge_index, W1, b1, W2, b2):
    src = edge_index[0]
    dst = edge_index[1]
    pad = jnp.full((EPAD - E,), N, jnp.int32)
    sidx = jnp.concatenate([src, pad]).reshape(NS * CPT, CHUNK)
    didx = jnp.concatenate([dst, pad]).reshape(NS * CPT, CHUNK)
    eidx = jnp.concatenate([sidx, didx], axis=0)
    eidx2 = jnp.stack([sidx, didx], axis=1)
    xp = jnp.zeros((NP, F_IN), jnp.float32).at[:N].set(x)

    rv = _get_degree_kernel()(eidx)
    h1 = _mm1(xp, W1, b1)
    z1 = _make_prop(HID)(h1, rv, eidx2)
    h2 = _mm2(z1, W2, b2)
    z2 = _make_prop(OUT, merge_out=True)(h2, rv, eidx2)
    return z2[:N]
